# Initial kernel scaffold; baseline (speedup 1.0000x reference)
import functools

import jax
import jax.numpy as jnp
from jax import lax
from jax.experimental import pallas as pl
from jax.experimental.pallas import tpu as pltpu
from jax.experimental.pallas import tpu_sc as plsc

N = 10000
E = 320000
C = 128

NC = 2
NS = 16
NW = NC * NS
CHUNK = 128
N_PAD = 10240
E_PAD = 327680
NCHUNKS = E_PAD // (NW * CHUNK)
ROWS_PER_TILE = N_PAD // NS


def _mesh():
    return plsc.VectorSubcoreMesh(core_axis_name="c", subcore_axis_name="s")


def _deg_body(dst_hbm, ew_hbm, out_hbm, dst_v, ew_v, zbuf, deg_sh):
    c = lax.axis_index("c")
    s = lax.axis_index("s")
    wid = s * NC + c

    pltpu.sync_copy(dst_hbm.at[wid], dst_v)
    pltpu.sync_copy(ew_hbm.at[wid], ew_v)

    def zstep(i, carry):
        zbuf[pl.ds(i * 16, 16)] = jnp.zeros((16,), jnp.float32)
        return carry

    lax.fori_loop(0, ROWS_PER_TILE // 16, zstep, 0)
    pltpu.sync_copy(zbuf, deg_sh.at[pl.ds(s * ROWS_PER_TILE, ROWS_PER_TILE)])
    plsc.subcore_barrier()

    def chunk(g, carry):
        pltpu.sync_copy(ew_v.at[g], deg_sh.at[dst_v.at[g]], add=True)
        return carry

    lax.fori_loop(0, NCHUNKS, chunk, 0)
    plsc.subcore_barrier()

    sl = pl.ds(s * ROWS_PER_TILE, ROWS_PER_TILE)
    pltpu.sync_copy(deg_sh.at[sl], out_hbm.at[c, sl])


def _deg_call(dst_r, ew_r):
    fn = pl.kernel(
        _deg_body,
        out_type=jax.ShapeDtypeStruct((NC, N_PAD), jnp.float32),
        mesh=_mesh(),
        scratch_types=[
            pltpu.VMEM((NCHUNKS, CHUNK), jnp.int32),
            pltpu.VMEM((NCHUNKS, CHUNK), jnp.float32),
            pltpu.VMEM((ROWS_PER_TILE,), jnp.float32),
            pltpu.VMEM_SHARED((N_PAD,), jnp.float32),
        ],
    )
    return fn(dst_r, ew_r)


def _edge_body(y_hbm, src_hbm, dst_hbm, ew_hbm, out_hbm,
               src_v, dst_v, ew_v, rows, acc_sh, gsem):
    c = lax.axis_index("c")
    s = lax.axis_index("s")
    wid = s * NC + c

    pltpu.sync_copy(src_hbm.at[wid], src_v)
    pltpu.sync_copy(dst_hbm.at[wid], dst_v)
    pltpu.sync_copy(ew_hbm.at[wid], ew_v)

    def zstep(i, carry):
        for j in range(C // 16):
            rows[i, pl.ds(j * 16, 16)] = jnp.zeros((16,), jnp.float32)
        return carry

    lax.fori_loop(0, CHUNK, zstep, 0)
    for k in range(ROWS_PER_TILE // CHUNK):
        pltpu.sync_copy(
            rows, acc_sh.at[pl.ds(s * ROWS_PER_TILE + k * CHUNK, CHUNK)])
    plsc.subcore_barrier()

    def chunk(g, carry):
        pltpu.async_copy(y_hbm.at[src_v.at[g]], rows, gsem).wait()

        def edge(e, carry2):
            w = ew_v[g, e]
            for j in range(C // 16):
                sl = pl.ds(j * 16, 16)
                rows[e, sl] = rows[e, sl] * w
            return carry2

        lax.fori_loop(0, CHUNK, edge, 0)
        pltpu.sync_copy(rows, acc_sh.at[dst_v.at[g]], add=True)
        return carry

    lax.fori_loop(0, NCHUNKS, chunk, 0)
    plsc.subcore_barrier()

    sl = pl.ds(s * ROWS_PER_TILE, ROWS_PER_TILE)
    pltpu.sync_copy(acc_sh.at[sl], out_hbm.at[c, sl])


def _edge_call(y, src_r, dst_r, ew_r):
    fn = pl.kernel(
        _edge_body,
        out_type=jax.ShapeDtypeStruct((NC, N_PAD, C), jnp.float32),
        mesh=_mesh(),
        scratch_types=[
            pltpu.VMEM((NCHUNKS, CHUNK), jnp.int32),
            pltpu.VMEM((NCHUNKS, CHUNK), jnp.int32),
            pltpu.VMEM((NCHUNKS, CHUNK), jnp.float32),
            pltpu.VMEM((CHUNK, C), jnp.float32),
            pltpu.VMEM_SHARED((N_PAD, C), jnp.float32),
            pltpu.SemaphoreType.DMA,
        ],
    )
    return fn(y, src_r, dst_r, ew_r)


def _tc_first_body(degp_ref, x_ref, w_ref, dis_ref, y_ref):
    deg = degp_ref[0] + degp_ref[1] + 1.0
    dis = jnp.where(deg > 0, lax.rsqrt(jnp.maximum(deg, 1e-12)), 0.0)
    dis_ref[...] = dis
    xw = jnp.dot(x_ref[...], w_ref[...], preferred_element_type=jnp.float32)
    y_ref[...] = xw * dis


def _tc_mid_body(accp_ref, y_ref, dis_ref, b_ref, w_ref, y2_ref):
    dis = dis_ref[...]
    pre = dis * (accp_ref[0] + accp_ref[1] + y_ref[...]) + b_ref[...]
    h = jnp.maximum(pre, 0.0)
    y2_ref[...] = jnp.dot(h, w_ref[...], preferred_element_type=jnp.float32) * dis


def _tc_last_body(accp_ref, y_ref, dis_ref, b_ref, out_ref):
    out_ref[...] = (dis_ref[...] * (accp_ref[0] + accp_ref[1] + y_ref[...])
                    + b_ref[...])


def kernel(x, edge_index, edge_weight, W1, b1, W2, b2):
    src = edge_index[0]
    dst = edge_index[1]
    pad_e = E_PAD - E
    src_r = jnp.concatenate(
        [src, jnp.zeros((pad_e,), src.dtype)]).reshape(NW, NCHUNKS, CHUNK)
    dst_r = jnp.concatenate(
        [dst, jnp.zeros((pad_e,), dst.dtype)]).reshape(NW, NCHUNKS, CHUNK)
    ew_r = jnp.concatenate(
        [edge_weight, jnp.zeros((pad_e,), edge_weight.dtype)]
    ).reshape(NW, NCHUNKS, CHUNK)
    x_pad = jnp.pad(x, ((0, N_PAD - N), (0, 0)))

    degp = _deg_call(dst_r, ew_r).reshape(NC, N_PAD, 1)

    tc_first = pl.pallas_call(
        _tc_first_body,
        out_shape=(
            jax.ShapeDtypeStruct((N_PAD, 1), jnp.float32),
            jax.ShapeDtypeStruct((N_PAD, C), jnp.float32),
        ),
    )
    dis, y1 = tc_first(degp, x_pad, W1)

    acc1 = _edge_call(y1, src_r, dst_r, ew_r)

    tc_mid = pl.pallas_call(
        _tc_mid_body,
        out_shape=jax.ShapeDtypeStruct((N_PAD, C), jnp.float32),
    )
    y2 = tc_mid(acc1, y1, dis, b1.reshape(1, C), W2)

    acc2 = _edge_call(y2, src_r, dst_r, ew_r)

    tc_last = pl.pallas_call(
        _tc_last_body,
        out_shape=jax.ShapeDtypeStruct((N_PAD, C), jnp.float32),
    )
    out = tc_last(acc2, y2, dis, b2.reshape(1, C))
    return out[:N]


# speedup vs baseline: 7.1459x; 7.1459x over previous
import functools

import jax
import jax.numpy as jnp
from jax import lax
from jax.experimental import pallas as pl
from jax.experimental.pallas import tpu as pltpu
from jax.experimental.pallas import tpu_sc as plsc

N = 10000
E = 320000
C = 128

NC = 2
NS = 16
NW = NC * NS
CHUNK = 128
N_PAD = 10240
E_PAD = 327680
NCHUNKS = E_PAD // (NW * CHUNK)
ROWS_PER_TILE = N_PAD // NS


def _mesh():
    return plsc.VectorSubcoreMesh(core_axis_name="c", subcore_axis_name="s")


def _deg_body(dst_hbm, ew_hbm, out_hbm, dst_v, ew_v, zbuf, deg_sh):
    c = lax.axis_index("c")
    s = lax.axis_index("s")
    wid = s * NC + c

    pltpu.sync_copy(dst_hbm.at[wid], dst_v)
    pltpu.sync_copy(ew_hbm.at[wid], ew_v)

    def zstep(i, carry):
        zbuf[pl.ds(i * 16, 16)] = jnp.zeros((16,), jnp.float32)
        return carry

    lax.fori_loop(0, ROWS_PER_TILE // 16, zstep, 0)
    pltpu.sync_copy(zbuf, deg_sh.at[pl.ds(s * ROWS_PER_TILE, ROWS_PER_TILE)])
    plsc.subcore_barrier()

    def chunk(g, carry):
        pltpu.sync_copy(ew_v.at[g], deg_sh.at[dst_v.at[g]], add=True)
        return carry

    lax.fori_loop(0, NCHUNKS, chunk, 0)
    plsc.subcore_barrier()

    sl = pl.ds(s * ROWS_PER_TILE, ROWS_PER_TILE)
    pltpu.sync_copy(deg_sh.at[sl], out_hbm.at[c, sl])


def _deg_call(dst_r, ew_r):
    fn = pl.kernel(
        _deg_body,
        out_type=jax.ShapeDtypeStruct((NC, N_PAD), jnp.float32),
        mesh=_mesh(),
        scratch_types=[
            pltpu.VMEM((NCHUNKS, CHUNK), jnp.int32),
            pltpu.VMEM((NCHUNKS, CHUNK), jnp.float32),
            pltpu.VMEM((ROWS_PER_TILE,), jnp.float32),
            pltpu.VMEM_SHARED((N_PAD,), jnp.float32),
        ],
    )
    return fn(dst_r, ew_r)


def _edge_body(y_hbm, src_hbm, dst_hbm, ew_hbm, out_hbm,
               src_v, dst_v, ew_v, rows, acc_sh, gsem):
    c = lax.axis_index("c")
    s = lax.axis_index("s")
    wid = s * NC + c

    pltpu.sync_copy(src_hbm.at[wid], src_v)
    pltpu.sync_copy(dst_hbm.at[wid], dst_v)
    pltpu.sync_copy(ew_hbm.at[wid], ew_v)

    def zstep(i, carry):
        for j in range(C // 16):
            rows[i, pl.ds(j * 16, 16)] = jnp.zeros((16,), jnp.float32)
        return carry

    lax.fori_loop(0, CHUNK, zstep, 0)
    for k in range(ROWS_PER_TILE // CHUNK):
        pltpu.sync_copy(
            rows, acc_sh.at[pl.ds(s * ROWS_PER_TILE + k * CHUNK, CHUNK)])
    plsc.subcore_barrier()

    def chunk(g, carry):
        pltpu.async_copy(y_hbm.at[src_v.at[g]], rows, gsem).wait()

        def edge16(eb, carry2):
            wvec = ew_v[g, pl.ds(eb * 16, 16)]
            for i in range(16):
                w = wvec[i]
                e = eb * 16 + i
                for j in range(C // 16):
                    sl = pl.ds(j * 16, 16)
                    rows[e, sl] = rows[e, sl] * w
            return carry2

        lax.fori_loop(0, CHUNK // 16, edge16, 0)
        pltpu.sync_copy(rows, acc_sh.at[dst_v.at[g]], add=True)
        return carry

    lax.fori_loop(0, NCHUNKS, chunk, 0)
    plsc.subcore_barrier()

    sl = pl.ds(s * ROWS_PER_TILE, ROWS_PER_TILE)
    pltpu.sync_copy(acc_sh.at[sl], out_hbm.at[c, sl])


def _edge_call(y, src_r, dst_r, ew_r):
    fn = pl.kernel(
        _edge_body,
        out_type=jax.ShapeDtypeStruct((NC, N_PAD, C), jnp.float32),
        mesh=_mesh(),
        scratch_types=[
            pltpu.VMEM((NCHUNKS, CHUNK), jnp.int32),
            pltpu.VMEM((NCHUNKS, CHUNK), jnp.int32),
            pltpu.VMEM((NCHUNKS, CHUNK), jnp.float32),
            pltpu.VMEM((CHUNK, C), jnp.float32),
            pltpu.VMEM_SHARED((N_PAD, C), jnp.float32),
            pltpu.SemaphoreType.DMA,
        ],
    )
    return fn(y, src_r, dst_r, ew_r)


def _tc_first_body(degp_ref, x_ref, w_ref, dis_ref, y_ref):
    deg = degp_ref[0] + degp_ref[1] + 1.0
    dis = jnp.where(deg > 0, lax.rsqrt(jnp.maximum(deg, 1e-12)), 0.0)
    dis_ref[...] = dis
    xw = jnp.dot(x_ref[...], w_ref[...], preferred_element_type=jnp.float32)
    y_ref[...] = xw * dis


def _tc_mid_body(accp_ref, y_ref, dis_ref, b_ref, w_ref, y2_ref):
    dis = dis_ref[...]
    pre = dis * (accp_ref[0] + accp_ref[1] + y_ref[...]) + b_ref[...]
    h = jnp.maximum(pre, 0.0)
    y2_ref[...] = jnp.dot(h, w_ref[...], preferred_element_type=jnp.float32) * dis


def _tc_last_body(accp_ref, y_ref, dis_ref, b_ref, out_ref):
    out_ref[...] = (dis_ref[...] * (accp_ref[0] + accp_ref[1] + y_ref[...])
                    + b_ref[...])


def kernel(x, edge_index, edge_weight, W1, b1, W2, b2):
    src = edge_index[0]
    dst = edge_index[1]
    pad_e = E_PAD - E
    src_r = jnp.concatenate(
        [src, jnp.zeros((pad_e,), src.dtype)]).reshape(NW, NCHUNKS, CHUNK)
    dst_r = jnp.concatenate(
        [dst, jnp.zeros((pad_e,), dst.dtype)]).reshape(NW, NCHUNKS, CHUNK)
    ew_r = jnp.concatenate(
        [edge_weight, jnp.zeros((pad_e,), edge_weight.dtype)]
    ).reshape(NW, NCHUNKS, CHUNK)
    x_pad = jnp.pad(x, ((0, N_PAD - N), (0, 0)))

    degp = _deg_call(dst_r, ew_r).reshape(NC, N_PAD, 1)

    tc_first = pl.pallas_call(
        _tc_first_body,
        out_shape=(
            jax.ShapeDtypeStruct((N_PAD, 1), jnp.float32),
            jax.ShapeDtypeStruct((N_PAD, C), jnp.float32),
        ),
    )
    dis, y1 = tc_first(degp, x_pad, W1)

    acc1 = _edge_call(y1, src_r, dst_r, ew_r)

    tc_mid = pl.pallas_call(
        _tc_mid_body,
        out_shape=jax.ShapeDtypeStruct((N_PAD, C), jnp.float32),
    )
    y2 = tc_mid(acc1, y1, dis, b1.reshape(1, C), W2)

    acc2 = _edge_call(y2, src_r, dst_r, ew_r)

    tc_last = pl.pallas_call(
        _tc_last_body,
        out_shape=jax.ShapeDtypeStruct((N_PAD, C), jnp.float32),
    )
    out = tc_last(acc2, y2, dis, b2.reshape(1, C))
    return out[:N]


# speedup vs baseline: 9.4726x; 1.3256x over previous
import functools

import jax
import jax.numpy as jnp
from jax import lax
from jax.experimental import pallas as pl
from jax.experimental.pallas import tpu as pltpu
from jax.experimental.pallas import tpu_sc as plsc

N = 10000
E = 320000
C = 128

NC = 2
NS = 16
NW = NC * NS
CHUNK = 128
N_PAD = 10240
E_PAD = 327680
NCHUNKS = E_PAD // (NW * CHUNK)
ROWS_PER_TILE = N_PAD // NS


def _mesh():
    return plsc.VectorSubcoreMesh(core_axis_name="c", subcore_axis_name="s")


def _deg_body(dst_hbm, ew_hbm, out_hbm, dst_v, ew_v, zbuf, deg_sh):
    c = lax.axis_index("c")
    s = lax.axis_index("s")
    wid = s * NC + c

    pltpu.sync_copy(dst_hbm.at[wid], dst_v)
    pltpu.sync_copy(ew_hbm.at[wid], ew_v)

    def zstep(i, carry):
        zbuf[pl.ds(i * 16, 16)] = jnp.zeros((16,), jnp.float32)
        return carry

    lax.fori_loop(0, ROWS_PER_TILE // 16, zstep, 0)
    pltpu.sync_copy(zbuf, deg_sh.at[pl.ds(s * ROWS_PER_TILE, ROWS_PER_TILE)])
    plsc.subcore_barrier()

    def chunk(g, carry):
        pltpu.sync_copy(ew_v.at[g], deg_sh.at[dst_v.at[g]], add=True)
        return carry

    lax.fori_loop(0, NCHUNKS, chunk, 0)
    plsc.subcore_barrier()

    sl = pl.ds(s * ROWS_PER_TILE, ROWS_PER_TILE)
    pltpu.sync_copy(deg_sh.at[sl], out_hbm.at[c, sl])


def _deg_call(dst_r, ew_r):
    fn = pl.kernel(
        _deg_body,
        out_type=jax.ShapeDtypeStruct((NC, N_PAD), jnp.float32),
        mesh=_mesh(),
        scratch_types=[
            pltpu.VMEM((NCHUNKS, CHUNK), jnp.int32),
            pltpu.VMEM((NCHUNKS, CHUNK), jnp.float32),
            pltpu.VMEM((ROWS_PER_TILE,), jnp.float32),
            pltpu.VMEM_SHARED((N_PAD,), jnp.float32),
        ],
    )
    return fn(dst_r, ew_r)


def _edge_body(y_hbm, idx_hbm, ew_hbm, out_hbm,
               ring, ew_v, gb0, gb1, acc_sh,
               gs0, gs1, is0, is1, is2, is3):
    c = lax.axis_index("c")
    s = lax.axis_index("s")
    wid = s * NC + c

    pltpu.sync_copy(ew_hbm.at[wid], ew_v)

    gb = (gb0, gb1)
    gs = (gs0, gs1)
    isem = (is0, is1, is2, is3)

    def zstep(i, carry):
        for j in range(C // 16):
            gb0[i, pl.ds(j * 16, 16)] = jnp.zeros((16,), jnp.float32)
        return carry

    lax.fori_loop(0, CHUNK, zstep, 0)
    for k in range(ROWS_PER_TILE // CHUNK):
        pltpu.sync_copy(
            gb0, acc_sh.at[pl.ds(s * ROWS_PER_TILE + k * CHUNK, CHUNK)])
    plsc.subcore_barrier()

    def i_start(g, r):
        pltpu.async_copy(idx_hbm.at[wid, g], ring.at[r], isem[r])

    def i_wait(g, r):
        pltpu.make_async_copy(idx_hbm.at[wid, g], ring.at[r], isem[r]).wait()

    def g_start(r, b):
        pltpu.async_copy(y_hbm.at[ring.at[r, 0]], gb[b], gs[b])

    def g_wait(r, b):
        pltpu.make_async_copy(y_hbm.at[ring.at[r, 0]], gb[b], gs[b]).wait()

    def scale(g, b):
        gbuf = gb[b]

        def edge16(eb, carry2):
            wvec = ew_v[g, pl.ds(eb * 16, 16)]
            for i in range(16):
                w = wvec[i]
                e = eb * 16 + i
                for j in range(C // 16):
                    sl = pl.ds(j * 16, 16)
                    gbuf[e, sl] = gbuf[e, sl] * w
            return carry2

        lax.fori_loop(0, CHUNK // 16, edge16, 0)

    def chunk_work(g, r, do_refill, do_issue):
        b = r % 2
        g_wait(r, b)
        scale(g, b)
        pltpu.sync_copy(gb[b], acc_sh.at[ring.at[r, 1]], add=True)
        if do_issue:
            i_wait(g + 2, (r + 2) % 4)
            g_start((r + 2) % 4, b)
        if do_refill:
            i_start(g + 4, r)

    for r in range(4):
        i_start(r, r)
    i_wait(0, 0)
    g_start(0, 0)
    i_wait(1, 1)
    g_start(1, 1)

    def grp_body(sg, carry):
        for r in range(4):
            chunk_work(sg * 4 + r, r, True, True)
        return carry

    lax.fori_loop(0, NCHUNKS // 4 - 1, grp_body, 0)

    for r in range(2):
        chunk_work(NCHUNKS - 4 + r, r, False, True)
    for r in range(2):
        chunk_work(NCHUNKS - 2 + r, r + 2, False, False)
    plsc.subcore_barrier()

    sl = pl.ds(s * ROWS_PER_TILE, ROWS_PER_TILE)
    pltpu.sync_copy(acc_sh.at[sl], out_hbm.at[c, sl])


def _edge_call(y, idx_r, ew_r):
    fn = pl.kernel(
        _edge_body,
        out_type=jax.ShapeDtypeStruct((NC, N_PAD, C), jnp.float32),
        mesh=_mesh(),
        scratch_types=[
            pltpu.VMEM((4, 2, CHUNK), jnp.int32),
            pltpu.VMEM((NCHUNKS, CHUNK), jnp.float32),
            pltpu.VMEM((CHUNK, C), jnp.float32),
            pltpu.VMEM((CHUNK, C), jnp.float32),
            pltpu.VMEM_SHARED((N_PAD, C), jnp.float32),
            pltpu.SemaphoreType.DMA,
            pltpu.SemaphoreType.DMA,
            pltpu.SemaphoreType.DMA,
            pltpu.SemaphoreType.DMA,
            pltpu.SemaphoreType.DMA,
            pltpu.SemaphoreType.DMA,
        ],
    )
    return fn(y, idx_r, ew_r)


def _tc_xw_body(x_ref, w_ref, xw_ref):
    xw_ref[...] = jnp.dot(x_ref[...], w_ref[...],
                          preferred_element_type=jnp.float32)


def _tc_first_body(degp_ref, xw_ref, dis_ref, y_ref):
    deg = degp_ref[0] + degp_ref[1] + 1.0
    dis = jnp.where(deg > 0, lax.rsqrt(jnp.maximum(deg, 1e-12)), 0.0)
    dis_ref[...] = dis
    y_ref[...] = xw_ref[...] * dis


def _tc_mid_body(accp_ref, y_ref, dis_ref, b_ref, w_ref, y2_ref):
    dis = dis_ref[...]
    pre = dis * (accp_ref[0] + accp_ref[1] + y_ref[...]) + b_ref[...]
    h = jnp.maximum(pre, 0.0)
    y2_ref[...] = jnp.dot(h, w_ref[...], preferred_element_type=jnp.float32) * dis


def _tc_last_body(accp_ref, y_ref, dis_ref, b_ref, out_ref):
    out_ref[...] = (dis_ref[...] * (accp_ref[0] + accp_ref[1] + y_ref[...])
                    + b_ref[...])


def kernel(x, edge_index, edge_weight, W1, b1, W2, b2):
    src = edge_index[0]
    dst = edge_index[1]
    pad_e = E_PAD - E
    src_r = jnp.concatenate(
        [src, jnp.zeros((pad_e,), src.dtype)]).reshape(NW, NCHUNKS, CHUNK)
    dst_r = jnp.concatenate(
        [dst, jnp.zeros((pad_e,), dst.dtype)]).reshape(NW, NCHUNKS, CHUNK)
    ew_r = jnp.concatenate(
        [edge_weight, jnp.zeros((pad_e,), edge_weight.dtype)]
    ).reshape(NW, NCHUNKS, CHUNK)
    idx_r = jnp.stack([src_r, dst_r], axis=2)
    x_pad = jnp.pad(x, ((0, N_PAD - N), (0, 0)))

    tc_xw = pl.pallas_call(
        _tc_xw_body,
        out_shape=jax.ShapeDtypeStruct((N_PAD, C), jnp.float32),
    )
    xw = tc_xw(x_pad, W1)
    degp = _deg_call(dst_r, ew_r).reshape(NC, N_PAD, 1)

    tc_first = pl.pallas_call(
        _tc_first_body,
        out_shape=(
            jax.ShapeDtypeStruct((N_PAD, 1), jnp.float32),
            jax.ShapeDtypeStruct((N_PAD, C), jnp.float32),
        ),
    )
    dis, y1 = tc_first(degp, xw)

    acc1 = _edge_call(y1, idx_r, ew_r)

    tc_mid = pl.pallas_call(
        _tc_mid_body,
        out_shape=jax.ShapeDtypeStruct((N_PAD, C), jnp.float32),
    )
    y2 = tc_mid(acc1, y1, dis, b1.reshape(1, C), W2)

    acc2 = _edge_call(y2, idx_r, ew_r)

    tc_last = pl.pallas_call(
        _tc_last_body,
        out_shape=jax.ShapeDtypeStruct((N_PAD, C), jnp.float32),
    )
    out = tc_last(acc2, y2, dis, b2.reshape(1, C))
    return out[:N]
